# baseline (speedup 1.0000x reference)
import functools
import math

import jax
import jax.numpy as jnp
from jax import lax
from jax.experimental import pallas as pl
from jax.experimental.pallas import tpu as pltpu
from jax.experimental.pallas import tpu_sc as plsc

NUM_EMB = 100000
EMB_DIM = 64
BATCH = 16384

NC = 2
NS = 16
NW = NC * NS
B_PER_W = BATCH // NW
CHUNK = 128
NCHUNK = B_PER_W // CHUNK

_INV_SQRT2 = 1.0 / math.sqrt(2.0)
_P_BLK = 4000
_S_BLK = 2048


def _proj_body(tab_ref, wt_ref, b_ref, out_ref):
    for half in (0, 1):
        rows = tab_ref[pl.Slice(half, _P_BLK // 2, 2), :]
        proj = jnp.dot(rows, wt_ref[...],
                       preferred_element_type=jnp.float32) + b_ref[...]
        act = 0.5 * proj * (1.0 + lax.erf(proj * _INV_SQRT2))
        out_ref[:, half * EMB_DIM:(half + 1) * EMB_DIM] = act


def _tc_project_table(table, wt, b2):
    return pl.pallas_call(
        _proj_body,
        grid=(NUM_EMB // _P_BLK,),
        in_specs=[
            pl.BlockSpec((_P_BLK, EMB_DIM), lambda i: (i, 0)),
            pl.BlockSpec((EMB_DIM, EMB_DIM), lambda i: (0, 0)),
            pl.BlockSpec((1, EMB_DIM), lambda i: (0, 0)),
        ],
        out_specs=pl.BlockSpec((_P_BLK // 2, 2 * EMB_DIM), lambda i: (i, 0)),
        out_shape=jax.ShapeDtypeStruct((NUM_EMB // 2, 2 * EMB_DIM), jnp.float32),
    )(table, wt, b2)


def _sc_gather_pairs(p2, pair_ids):
    mesh = plsc.VectorSubcoreMesh(core_axis_name="c", subcore_axis_name="s")

    @functools.partial(
        pl.kernel,
        out_type=jax.ShapeDtypeStruct((BATCH, 2 * EMB_DIM), jnp.float32),
        mesh=mesh,
        scratch_types=[
            pltpu.VMEM((B_PER_W,), jnp.int32),
            pltpu.VMEM((B_PER_W, 2 * EMB_DIM), jnp.float32),
            pltpu.SemaphoreType.DMA,
        ],
        compiler_params=pltpu.CompilerParams(use_tc_tiling_on_sc=True),
    )
    def k(p2_hbm, idx_hbm, out_hbm, idx_v, rows_v, sem):
        wid = lax.axis_index("s") * NC + lax.axis_index("c")
        base = wid * B_PER_W
        pltpu.sync_copy(idx_hbm.at[pl.ds(base, B_PER_W)], idx_v)
        copies = []
        for j in range(NCHUNK):
            copies.append(
                pltpu.async_copy(
                    p2_hbm.at[idx_v.at[pl.ds(j * CHUNK, CHUNK)]],
                    rows_v.at[pl.ds(j * CHUNK, CHUNK)],
                    sem,
                )
            )
        for c in copies:
            c.wait()
        pltpu.sync_copy(rows_v, out_hbm.at[pl.ds(base, B_PER_W)])

    return k(p2, pair_ids)


def _sel_body(pairs_ref, par_ref, out_ref):
    pairs = pairs_ref[...]
    odd = par_ref[...] == 1
    out_ref[...] = jnp.where(odd, pairs[:, EMB_DIM:], pairs[:, :EMB_DIM])


def _tc_select_half(pairs, parity):
    return pl.pallas_call(
        _sel_body,
        grid=(BATCH // _S_BLK,),
        in_specs=[
            pl.BlockSpec((_S_BLK, 2 * EMB_DIM), lambda i: (i, 0)),
            pl.BlockSpec((_S_BLK, 1), lambda i: (i, 0)),
        ],
        out_specs=pl.BlockSpec((_S_BLK, EMB_DIM), lambda i: (i, 0)),
        out_shape=jax.ShapeDtypeStruct((BATCH, EMB_DIM), jnp.float32),
    )(pairs, parity)


def kernel(country_ids, table, W, b):
    ids = country_ids.astype(jnp.int32)
    pair_ids = jnp.right_shift(ids, 1)
    parity = jnp.bitwise_and(ids, 1).reshape(BATCH, 1)
    p2 = _tc_project_table(table, W.T, b.reshape(1, EMB_DIM))
    pairs = _sc_gather_pairs(p2, pair_ids)
    return _tc_select_half(pairs, parity)


# speedup vs baseline: 1.6026x; 1.6026x over previous
import functools
import math

import jax
import jax.numpy as jnp
from jax import lax
from jax.experimental import pallas as pl
from jax.experimental.pallas import tpu as pltpu
from jax.experimental.pallas import tpu_sc as plsc

NUM_EMB = 100000
EMB_DIM = 64
BATCH = 16384

NC = 2
NS = 16
NW = NC * NS
B_PER_W = BATCH // NW
CHUNK = 128
NCHUNK = B_PER_W // CHUNK

_INV_SQRT2 = 1.0 / math.sqrt(2.0)
_H = 51200
_J_BLK = 2048
_S_BLK = 2048


def _proj_body(a_ref, b_ref, w_ref, bias_ref, out_ref):
    for half, ref in ((0, a_ref), (1, b_ref)):
        proj = jnp.dot(w_ref[...], ref[...],
                       preferred_element_type=jnp.float32) + bias_ref[...]
        act = 0.5 * proj * (1.0 + lax.erf(proj * _INV_SQRT2))
        out_ref[:, half * EMB_DIM:(half + 1) * EMB_DIM] = lax.transpose(act, (1, 0))


def _tc_project_table(tt, w, b_col):
    nj = _H // _J_BLK
    last = (NUM_EMB - 1) // _J_BLK
    return pl.pallas_call(
        _proj_body,
        grid=(nj,),
        in_specs=[
            pl.BlockSpec((EMB_DIM, _J_BLK), lambda j: (0, j)),
            pl.BlockSpec(
                (EMB_DIM, _J_BLK),
                lambda j, nj=nj, last=last: (0, jnp.minimum(j + nj, last)),
            ),
            pl.BlockSpec((EMB_DIM, EMB_DIM), lambda j: (0, 0)),
            pl.BlockSpec((EMB_DIM, 1), lambda j: (0, 0)),
        ],
        out_specs=pl.BlockSpec((_J_BLK, 2 * EMB_DIM), lambda j: (j, 0)),
        out_shape=jax.ShapeDtypeStruct((_H, 2 * EMB_DIM), jnp.float32),
    )(tt, tt, w, b_col)


def _sc_gather_pairs(p2, fold_ids):
    mesh = plsc.VectorSubcoreMesh(core_axis_name="c", subcore_axis_name="s")

    @functools.partial(
        pl.kernel,
        out_type=jax.ShapeDtypeStruct((BATCH, 2 * EMB_DIM), jnp.float32),
        mesh=mesh,
        scratch_types=[
            pltpu.VMEM((B_PER_W,), jnp.int32),
            pltpu.VMEM((B_PER_W, 2 * EMB_DIM), jnp.float32),
            pltpu.SemaphoreType.DMA,
        ],
        compiler_params=pltpu.CompilerParams(use_tc_tiling_on_sc=True),
    )
    def k(p2_hbm, idx_hbm, out_hbm, idx_v, rows_v, sem):
        wid = lax.axis_index("s") * NC + lax.axis_index("c")
        base = wid * B_PER_W
        pltpu.sync_copy(idx_hbm.at[pl.ds(base, B_PER_W)], idx_v)
        copies = []
        for j in range(NCHUNK):
            copies.append(
                pltpu.async_copy(
                    p2_hbm.at[idx_v.at[pl.ds(j * CHUNK, CHUNK)]],
                    rows_v.at[pl.ds(j * CHUNK, CHUNK)],
                    sem,
                )
            )
        for c in copies:
            c.wait()
        pltpu.sync_copy(rows_v, out_hbm.at[pl.ds(base, B_PER_W)])

    return k(p2, fold_ids)


def _sel_body(pairs_ref, par_ref, out_ref):
    p = pairs_ref[...]
    left = lax.transpose(p[:, :EMB_DIM], (1, 0))
    right = lax.transpose(p[:, EMB_DIM:], (1, 0))
    out_ref[...] = jnp.where(par_ref[...] == 1, right, left)


def _tc_select_half(pairs, parity_row):
    return pl.pallas_call(
        _sel_body,
        grid=(BATCH // _S_BLK,),
        in_specs=[
            pl.BlockSpec((_S_BLK, 2 * EMB_DIM), lambda i: (i, 0)),
            pl.BlockSpec((1, _S_BLK), lambda i: (0, i)),
        ],
        out_specs=pl.BlockSpec((EMB_DIM, _S_BLK), lambda i: (0, i)),
        out_shape=jax.ShapeDtypeStruct((EMB_DIM, BATCH), jnp.float32),
    )(pairs, parity_row)


def kernel(country_ids, table, W, b):
    ids = country_ids.astype(jnp.int32)
    fold_ids = jnp.where(ids < _H, ids, ids - _H)
    parity_row = (ids >= _H).astype(jnp.int32).reshape(1, BATCH)
    p2 = _tc_project_table(table.T, W, b.reshape(EMB_DIM, 1))
    pairs = _sc_gather_pairs(p2, fold_ids)
    return _tc_select_half(pairs, parity_row).T


# speedup vs baseline: 1.6462x; 1.0272x over previous
import functools
import math

import jax
import jax.numpy as jnp
from jax import lax
from jax.experimental import pallas as pl
from jax.experimental.pallas import tpu as pltpu
from jax.experimental.pallas import tpu_sc as plsc

NUM_EMB = 100000
EMB_DIM = 64
BATCH = 16384

NC = 2
NS = 16
NW = NC * NS
B_PER_W = BATCH // NW
CHUNK = 128
NCHUNK = B_PER_W // CHUNK

_INV_SQRT2 = 1.0 / math.sqrt(2.0)
_H = 51200
_J_BLK = 2048
_S_BLK = 2048


def _proj_body(a_ref, b_ref, w_ref, bias_ref, out_ref):
    for half, ref in ((0, a_ref), (1, b_ref)):
        proj = lax.dot_general(ref[...], w_ref[...], (((0,), (1,)), ((), ())),
                               preferred_element_type=jnp.float32) + bias_ref[...]
        act = 0.5 * proj * (1.0 + lax.erf(proj * _INV_SQRT2))
        out_ref[:, half * EMB_DIM:(half + 1) * EMB_DIM] = act


def _tc_project_table(tt, w, b_col):
    nj = _H // _J_BLK
    last = (NUM_EMB - 1) // _J_BLK
    return pl.pallas_call(
        _proj_body,
        grid=(nj,),
        in_specs=[
            pl.BlockSpec((EMB_DIM, _J_BLK), lambda j: (0, j)),
            pl.BlockSpec(
                (EMB_DIM, _J_BLK),
                lambda j, nj=nj, last=last: (0, jnp.minimum(j + nj, last)),
            ),
            pl.BlockSpec((EMB_DIM, EMB_DIM), lambda j: (0, 0)),
            pl.BlockSpec((1, EMB_DIM), lambda j: (0, 0)),
        ],
        out_specs=pl.BlockSpec((_J_BLK, 2 * EMB_DIM), lambda j: (j, 0)),
        out_shape=jax.ShapeDtypeStruct((_H, 2 * EMB_DIM), jnp.float32),
    )(tt, tt, w, b_col)


def _sc_gather_pairs(p2, fold_ids):
    mesh = plsc.VectorSubcoreMesh(core_axis_name="c", subcore_axis_name="s")

    @functools.partial(
        pl.kernel,
        out_type=jax.ShapeDtypeStruct((BATCH, 2 * EMB_DIM), jnp.float32),
        mesh=mesh,
        scratch_types=[
            pltpu.VMEM((B_PER_W,), jnp.int32),
            pltpu.VMEM((B_PER_W, 2 * EMB_DIM), jnp.float32),
            pltpu.SemaphoreType.DMA,
        ],
        compiler_params=pltpu.CompilerParams(use_tc_tiling_on_sc=True),
    )
    def k(p2_hbm, idx_hbm, out_hbm, idx_v, rows_v, sem):
        wid = lax.axis_index("s") * NC + lax.axis_index("c")
        base = wid * B_PER_W
        pltpu.sync_copy(idx_hbm.at[pl.ds(base, B_PER_W)], idx_v)
        copies = []
        for j in range(NCHUNK):
            copies.append(
                pltpu.async_copy(
                    p2_hbm.at[idx_v.at[pl.ds(j * CHUNK, CHUNK)]],
                    rows_v.at[pl.ds(j * CHUNK, CHUNK)],
                    sem,
                )
            )
        for c in copies:
            c.wait()
        pltpu.sync_copy(rows_v, out_hbm.at[pl.ds(base, B_PER_W)])

    return k(p2, fold_ids)


def _sel_body(pairs_ref, par_ref, eye_ref, out_ref):
    p = pairs_ref[...]
    parc = lax.transpose(par_ref[...], (1, 0))
    sel = jnp.where(parc == 1, p[:, EMB_DIM:], p[:, :EMB_DIM])
    out_ref[...] = lax.dot_general(eye_ref[...], sel, (((1,), (1,)), ((), ())),
                                   preferred_element_type=jnp.float32)


def _tc_select_half(pairs, parity_row, eye):
    return pl.pallas_call(
        _sel_body,
        grid=(BATCH // _S_BLK,),
        in_specs=[
            pl.BlockSpec((_S_BLK, 2 * EMB_DIM), lambda i: (i, 0)),
            pl.BlockSpec((1, _S_BLK), lambda i: (0, i)),
            pl.BlockSpec((EMB_DIM, EMB_DIM), lambda i: (0, 0)),
        ],
        out_specs=pl.BlockSpec((EMB_DIM, _S_BLK), lambda i: (0, i)),
        out_shape=jax.ShapeDtypeStruct((EMB_DIM, BATCH), jnp.float32),
    )(pairs, parity_row, eye)


def kernel(country_ids, table, W, b):
    ids = country_ids.astype(jnp.int32)
    fold_ids = jnp.where(ids < _H, ids, ids - _H)
    parity_row = (ids >= _H).astype(jnp.int32).reshape(1, BATCH)
    p2 = _tc_project_table(table.T, W, b.reshape(1, EMB_DIM))
    pairs = _sc_gather_pairs(p2, fold_ids)
    eye = jnp.eye(EMB_DIM, dtype=jnp.float32)
    return _tc_select_half(pairs, parity_row, eye).T


# speedup vs baseline: 1.7643x; 1.0718x over previous
import functools
import math

import jax
import jax.numpy as jnp
from jax import lax
from jax.experimental import pallas as pl
from jax.experimental.pallas import tpu as pltpu
from jax.experimental.pallas import tpu_sc as plsc

NUM_EMB = 100000
EMB_DIM = 64
BATCH = 16384

NC = 2
NS = 16
NW = NC * NS
B_PER_W = BATCH // NW
CHUNK = 128
NCHUNK = B_PER_W // CHUNK

_INV_SQRT2 = 1.0 / math.sqrt(2.0)
_Q = 25600
_J_BLK = 2560
_S_BLK = 2048


def _bf16_bits(act):
    bits = lax.bitcast_convert_type(act, jnp.uint32)
    return (bits + jnp.uint32(0x7FFF) + ((bits >> 16) & jnp.uint32(1))) >> 16


def _proj_body(a_ref, b_ref, c_ref, d_ref, w_ref, bias_ref, out_ref):
    acts = []
    for ref in (a_ref, b_ref, c_ref, d_ref):
        proj = lax.dot_general(ref[...], w_ref[...], (((0,), (1,)), ((), ())),
                               preferred_element_type=jnp.float32) + bias_ref[...]
        acts.append(0.5 * proj * (1.0 + lax.erf(proj * _INV_SQRT2)))
    packed01 = _bf16_bits(acts[0]) | (_bf16_bits(acts[1]) << 16)
    packed23 = _bf16_bits(acts[2]) | (_bf16_bits(acts[3]) << 16)
    out_ref[:, :EMB_DIM] = packed01
    out_ref[:, EMB_DIM:] = packed23


def _tc_project_table(tt, w, b_col):
    nj = _Q // _J_BLK
    return pl.pallas_call(
        _proj_body,
        grid=(nj,),
        in_specs=[
            pl.BlockSpec((EMB_DIM, _J_BLK), lambda j, q=q, nj=nj: (0, j + q * nj))
            for q in range(4)
        ] + [
            pl.BlockSpec((EMB_DIM, EMB_DIM), lambda j: (0, 0)),
            pl.BlockSpec((1, EMB_DIM), lambda j: (0, 0)),
        ],
        out_specs=pl.BlockSpec((_J_BLK, 2 * EMB_DIM), lambda j: (j, 0)),
        out_shape=jax.ShapeDtypeStruct((_Q, 2 * EMB_DIM), jnp.uint32),
    )(tt, tt, tt, tt, w, b_col)


def _sc_gather_quads(p4, fold_ids):
    mesh = plsc.VectorSubcoreMesh(core_axis_name="c", subcore_axis_name="s")

    @functools.partial(
        pl.kernel,
        out_type=jax.ShapeDtypeStruct((BATCH, 2 * EMB_DIM), jnp.uint32),
        mesh=mesh,
        scratch_types=[
            pltpu.VMEM((B_PER_W,), jnp.int32),
            pltpu.VMEM((B_PER_W, 2 * EMB_DIM), jnp.uint32),
            pltpu.SemaphoreType.DMA,
        ],
        compiler_params=pltpu.CompilerParams(use_tc_tiling_on_sc=True),
    )
    def k(p4_hbm, idx_hbm, out_hbm, idx_v, rows_v, sem):
        wid = lax.axis_index("s") * NC + lax.axis_index("c")
        base = wid * B_PER_W
        pltpu.sync_copy(idx_hbm.at[pl.ds(base, B_PER_W)], idx_v)
        copies = []
        for j in range(NCHUNK):
            copies.append(
                pltpu.async_copy(
                    p4_hbm.at[idx_v.at[pl.ds(j * CHUNK, CHUNK)]],
                    rows_v.at[pl.ds(j * CHUNK, CHUNK)],
                    sem,
                )
            )
        for c in copies:
            c.wait()
        pltpu.sync_copy(rows_v, out_hbm.at[pl.ds(base, B_PER_W)])

    return k(p4, fold_ids)


def _sel_body(quads_ref, q_ref, eye_ref, out_ref):
    p = quads_ref[...]
    qc = lax.transpose(q_ref[...], (1, 0))
    u = jnp.where(qc >= 2, p[:, EMB_DIM:], p[:, :EMB_DIM])
    f32_bits = jnp.where((qc & 1) == 1,
                         u & jnp.uint32(0xFFFF0000),
                         u << 16)
    sel = lax.bitcast_convert_type(f32_bits, jnp.float32)
    out_ref[...] = lax.dot_general(eye_ref[...], sel, (((1,), (1,)), ((), ())),
                                   preferred_element_type=jnp.float32)


def _tc_select_quarter(quads, q_row, eye):
    return pl.pallas_call(
        _sel_body,
        grid=(BATCH // _S_BLK,),
        in_specs=[
            pl.BlockSpec((_S_BLK, 2 * EMB_DIM), lambda i: (i, 0)),
            pl.BlockSpec((1, _S_BLK), lambda i: (0, i)),
            pl.BlockSpec((EMB_DIM, EMB_DIM), lambda i: (0, 0)),
        ],
        out_specs=pl.BlockSpec((EMB_DIM, _S_BLK), lambda i: (0, i)),
        out_shape=jax.ShapeDtypeStruct((EMB_DIM, BATCH), jnp.float32),
    )(quads, q_row, eye)


def kernel(country_ids, table, W, b):
    ids = country_ids.astype(jnp.int32)
    q = ids // _Q
    fold_ids = ids - q * _Q
    q_row = q.reshape(1, BATCH)
    p4 = _tc_project_table(table.T, W, b.reshape(1, EMB_DIM))
    quads = _sc_gather_quads(p4, fold_ids)
    eye = jnp.eye(EMB_DIM, dtype=jnp.float32)
    return _tc_select_quarter(quads, q_row, eye).T


# speedup vs baseline: 1.8634x; 1.0562x over previous
import functools
import math

import jax
import jax.numpy as jnp
from jax import lax
from jax.experimental import pallas as pl
from jax.experimental.pallas import tpu as pltpu
from jax.experimental.pallas import tpu_sc as plsc

NUM_EMB = 100000
EMB_DIM = 64
BATCH = 16384

NC = 2
NS = 16
NW = NC * NS
B_PER_W = BATCH // NW
CHUNK = 128
NCHUNK = B_PER_W // CHUNK

_INV_SQRT2 = 1.0 / math.sqrt(2.0)
_Q = 25600
_J_BLK = 2560
_S_BLK = 2048


def _bf16_bits(act):
    bits = lax.bitcast_convert_type(act, jnp.uint32)
    return (bits + jnp.uint32(0x7FFF) + ((bits >> 16) & jnp.uint32(1))) >> 16


def _proj_body(a_ref, b_ref, c_ref, d_ref, w_ref, bias_ref, out_ref):
    projs = []
    for ref in (a_ref, b_ref, c_ref, d_ref):
        projs.append(
            lax.dot_general(ref[...], w_ref[...], (((0,), (1,)), ((), ())),
                            preferred_element_type=jnp.float32) + bias_ref[...])
    packed01 = _bf16_bits(projs[0]) | (_bf16_bits(projs[1]) << 16)
    packed23 = _bf16_bits(projs[2]) | (_bf16_bits(projs[3]) << 16)
    out_ref[:, :EMB_DIM] = packed01
    out_ref[:, EMB_DIM:] = packed23


def _tc_project_table(tt, w, b_col):
    nj = _Q // _J_BLK
    return pl.pallas_call(
        _proj_body,
        grid=(nj,),
        in_specs=[
            pl.BlockSpec((EMB_DIM, _J_BLK), lambda j, q=q, nj=nj: (0, j + q * nj))
            for q in range(4)
        ] + [
            pl.BlockSpec((EMB_DIM, EMB_DIM), lambda j: (0, 0)),
            pl.BlockSpec((1, EMB_DIM), lambda j: (0, 0)),
        ],
        out_specs=pl.BlockSpec((_J_BLK, 2 * EMB_DIM), lambda j: (j, 0)),
        out_shape=jax.ShapeDtypeStruct((_Q, 2 * EMB_DIM), jnp.uint32),
    )(tt, tt, tt, tt, w, b_col)


def _sc_gather_quads(p4, fold_ids):
    mesh = plsc.VectorSubcoreMesh(core_axis_name="c", subcore_axis_name="s")

    @functools.partial(
        pl.kernel,
        out_type=jax.ShapeDtypeStruct((BATCH, 2 * EMB_DIM), jnp.uint32),
        mesh=mesh,
        scratch_types=[
            pltpu.VMEM((B_PER_W,), jnp.int32),
            pltpu.VMEM((B_PER_W, 2 * EMB_DIM), jnp.uint32),
            pltpu.SemaphoreType.DMA,
        ],
        compiler_params=pltpu.CompilerParams(use_tc_tiling_on_sc=True),
    )
    def k(p4_hbm, idx_hbm, out_hbm, idx_v, rows_v, sem):
        wid = lax.axis_index("s") * NC + lax.axis_index("c")
        base = wid * B_PER_W
        pltpu.sync_copy(idx_hbm.at[pl.ds(base, B_PER_W)], idx_v)
        copies = []
        for j in range(NCHUNK):
            copies.append(
                pltpu.async_copy(
                    p4_hbm.at[idx_v.at[pl.ds(j * CHUNK, CHUNK)]],
                    rows_v.at[pl.ds(j * CHUNK, CHUNK)],
                    sem,
                )
            )
        for c in copies:
            c.wait()
        pltpu.sync_copy(rows_v, out_hbm.at[pl.ds(base, B_PER_W)])

    return k(p4, fold_ids)


def _sel_body(quads_ref, q_ref, eye_ref, out_ref):
    p = quads_ref[...]
    qc = lax.transpose(q_ref[...], (1, 0))
    u = jnp.where(qc >= 2, p[:, EMB_DIM:], p[:, :EMB_DIM])
    f32_bits = jnp.where((qc & 1) == 1,
                         u & jnp.uint32(0xFFFF0000),
                         u << 16)
    proj = lax.bitcast_convert_type(f32_bits, jnp.float32)
    act = 0.5 * proj * (1.0 + lax.erf(proj * _INV_SQRT2))
    out_ref[...] = lax.dot_general(eye_ref[...], act, (((1,), (1,)), ((), ())),
                                   preferred_element_type=jnp.float32)


def _tc_select_quarter(quads, q_row, eye):
    return pl.pallas_call(
        _sel_body,
        grid=(BATCH // _S_BLK,),
        in_specs=[
            pl.BlockSpec((_S_BLK, 2 * EMB_DIM), lambda i: (i, 0)),
            pl.BlockSpec((1, _S_BLK), lambda i: (0, i)),
            pl.BlockSpec((EMB_DIM, EMB_DIM), lambda i: (0, 0)),
        ],
        out_specs=pl.BlockSpec((EMB_DIM, _S_BLK), lambda i: (0, i)),
        out_shape=jax.ShapeDtypeStruct((EMB_DIM, BATCH), jnp.float32),
    )(quads, q_row, eye)


def kernel(country_ids, table, W, b):
    ids = country_ids.astype(jnp.int32)
    q = ids // _Q
    fold_ids = ids - q * _Q
    q_row = q.reshape(1, BATCH)
    p4 = _tc_project_table(table.T, W, b.reshape(1, EMB_DIM))
    quads = _sc_gather_quads(p4, fold_ids)
    eye = jnp.eye(EMB_DIM, dtype=jnp.float32)
    return _tc_select_quarter(quads, q_row, eye).T


# speedup vs baseline: 1.9126x; 1.0264x over previous
import functools
import math

import jax
import jax.numpy as jnp
from jax import lax
from jax.experimental import pallas as pl
from jax.experimental.pallas import tpu as pltpu
from jax.experimental.pallas import tpu_sc as plsc

NUM_EMB = 100000
EMB_DIM = 64
BATCH = 16384

NC = 2
NS = 16
NW = NC * NS
B_PER_W = BATCH // NW
CHUNK = 128
NCHUNK = B_PER_W // CHUNK

_INV_SQRT2 = 1.0 / math.sqrt(2.0)
_Q = 25600
_J_BLK = 5120
_S_BLK = 4096


def _bf16_bits(act):
    bits = lax.bitcast_convert_type(act, jnp.uint32)
    return (bits + jnp.uint32(0x7FFF) + ((bits >> 16) & jnp.uint32(1))) >> 16


def _proj_body(a_ref, b_ref, c_ref, d_ref, w_ref, bias_ref, out_ref):
    projs = []
    for ref in (a_ref, b_ref, c_ref, d_ref):
        projs.append(
            lax.dot_general(ref[...], w_ref[...], (((0,), (1,)), ((), ())),
                            preferred_element_type=jnp.float32) + bias_ref[...])
    packed01 = _bf16_bits(projs[0]) | (_bf16_bits(projs[1]) << 16)
    packed23 = _bf16_bits(projs[2]) | (_bf16_bits(projs[3]) << 16)
    out_ref[:, :EMB_DIM] = packed01
    out_ref[:, EMB_DIM:] = packed23


def _tc_project_table(tt, w, b_col):
    nj = _Q // _J_BLK
    return pl.pallas_call(
        _proj_body,
        grid=(nj,),
        in_specs=[
            pl.BlockSpec((EMB_DIM, _J_BLK), lambda j, q=q, nj=nj: (0, j + q * nj))
            for q in range(4)
        ] + [
            pl.BlockSpec((EMB_DIM, EMB_DIM), lambda j: (0, 0)),
            pl.BlockSpec((1, EMB_DIM), lambda j: (0, 0)),
        ],
        out_specs=pl.BlockSpec((_J_BLK, 2 * EMB_DIM), lambda j: (j, 0)),
        out_shape=jax.ShapeDtypeStruct((_Q, 2 * EMB_DIM), jnp.uint32),
    )(tt, tt, tt, tt, w, b_col)


def _sc_gather_quads(p4, fold_ids):
    mesh = plsc.VectorSubcoreMesh(core_axis_name="c", subcore_axis_name="s")

    @functools.partial(
        pl.kernel,
        out_type=jax.ShapeDtypeStruct((BATCH, 2 * EMB_DIM), jnp.uint32),
        mesh=mesh,
        scratch_types=[
            pltpu.VMEM((B_PER_W,), jnp.int32),
            pltpu.VMEM((B_PER_W, 2 * EMB_DIM), jnp.uint32),
            pltpu.SemaphoreType.DMA,
        ],
        compiler_params=pltpu.CompilerParams(use_tc_tiling_on_sc=True),
    )
    def k(p4_hbm, idx_hbm, out_hbm, idx_v, rows_v, sem):
        wid = lax.axis_index("s") * NC + lax.axis_index("c")
        base = wid * B_PER_W
        pltpu.sync_copy(idx_hbm.at[pl.ds(base, B_PER_W)], idx_v)
        copies = []
        for j in range(NCHUNK):
            copies.append(
                pltpu.async_copy(
                    p4_hbm.at[idx_v.at[pl.ds(j * CHUNK, CHUNK)]],
                    rows_v.at[pl.ds(j * CHUNK, CHUNK)],
                    sem,
                )
            )
        for c in copies:
            c.wait()
        pltpu.sync_copy(rows_v, out_hbm.at[pl.ds(base, B_PER_W)])

    return k(p4, fold_ids)


def _sel_body(quads_ref, q_ref, eye_ref, out_ref):
    p = quads_ref[...]
    qc = lax.transpose(q_ref[...], (1, 0))
    u = jnp.where(qc >= 2, p[:, EMB_DIM:], p[:, :EMB_DIM])
    f32_bits = jnp.where((qc & 1) == 1,
                         u & jnp.uint32(0xFFFF0000),
                         u << 16)
    proj = lax.bitcast_convert_type(f32_bits, jnp.float32)
    act = 0.5 * proj * (1.0 + lax.erf(proj * _INV_SQRT2))
    out_ref[...] = lax.dot_general(eye_ref[...], act, (((1,), (1,)), ((), ())),
                                   preferred_element_type=jnp.float32)


def _tc_select_quarter(quads, q_row, eye):
    return pl.pallas_call(
        _sel_body,
        grid=(BATCH // _S_BLK,),
        in_specs=[
            pl.BlockSpec((_S_BLK, 2 * EMB_DIM), lambda i: (i, 0)),
            pl.BlockSpec((1, _S_BLK), lambda i: (0, i)),
            pl.BlockSpec((EMB_DIM, EMB_DIM), lambda i: (0, 0)),
        ],
        out_specs=pl.BlockSpec((EMB_DIM, _S_BLK), lambda i: (0, i)),
        out_shape=jax.ShapeDtypeStruct((EMB_DIM, BATCH), jnp.float32),
    )(quads, q_row, eye)


def kernel(country_ids, table, W, b):
    ids = country_ids.astype(jnp.int32)
    q = ids // _Q
    fold_ids = ids - q * _Q
    q_row = q.reshape(1, BATCH)
    p4 = _tc_project_table(table.T, W, b.reshape(1, EMB_DIM))
    quads = _sc_gather_quads(p4, fold_ids)
    eye = jnp.eye(EMB_DIM, dtype=jnp.float32)
    return _tc_select_quarter(quads, q_row, eye).T
